# baseline (speedup 1.0000x reference)
import jax
import jax.numpy as jnp
from jax import lax
from jax.experimental import pallas as pl
from jax.experimental.pallas import tpu as pltpu
from jax.experimental.pallas import tpu_sc as plsc

N, C, H, W = 64, 128, 64, 64
FH, FW = 32, 32
NC, NS, L = 2, 16, 16
NW = NC * NS
CB = 16
NCHUNK = C // CB
IMG_W = C * H * W
CHUNK_W = CB * H * W
FCHUNK_W = CB * FH * FW
NG = N // L


def _lanes():
    return lax.iota(jnp.int32, L)


def _bcast(x):
    return lax.broadcast(jnp.int32(x) if isinstance(x, int) else x, (L,))


def _extract(vec, lane):
    return jnp.sum(jnp.where(_lanes() == lane, vec, jnp.int32(0)))


def _body(front_h, back_h, rois_h, out_h, rois_v, fbuf, buf):
    wid = lax.axis_index("s") * NC + lax.axis_index("c")
    pltpu.sync_copy(rois_h, rois_v)
    lanes = _lanes()

    face, facei = [], []
    for g in range(NG):
        base = (lanes + g * L) * 4
        x1c = plsc.load_gather(rois_v, [base])
        x2c = plsc.load_gather(rois_v, [base + 2])
        f = (x1c != 0) | (x2c != 0)
        face.append(f)
        facei.append(jnp.where(f, jnp.int32(1), jnp.int32(0)))
    counts = [jnp.sum(fi) for fi in facei]
    nf_total = counts[0]
    for g in range(1, NG):
        nf_total = nf_total + counts[g]
    ranks = []
    cf = jnp.int32(0)
    cn = jnp.int32(0)
    for g in range(NG):
        exclf = plsc.cumsum(facei[g]) - facei[g]
        nfi = 1 - facei[g]
        excln = plsc.cumsum(nfi) - nfi
        ranks.append(jnp.where(face[g], cf + exclf, nf_total + cn + excln))
        cf = cf + counts[g]
        cn = cn + (L - counts[g])

    grp = (2 * wid) // L
    grpv = _bcast(grp)
    rank_sel = ranks[NG - 1]
    for g in range(NG - 2, -1, -1):
        rank_sel = jnp.where(grpv == g, ranks[g], rank_sel)
    colbase = (lanes + grp * L) * 4
    col = [plsc.load_gather(rois_v, [colbase + c]) for c in range(4)]
    l0 = 2 * wid - grp * L

    params = []
    for img in range(2):
        ln = l0 + img
        x1 = _extract(col[0], ln)
        y1 = _extract(col[1], ln)
        x2 = _extract(col[2], ln)
        y2 = _extract(col[3], ln)
        rank = _extract(rank_sel, ln)
        h = y2 - y1
        hs = jnp.maximum(h, 1)
        ws = jnp.maximum(x2 - x1, 1)
        ixs, gact = [], []
        for g in range(W // L):
            xg = lanes + g * L
            t = jnp.maximum(xg - x1, 0) * FW
            ixs.append(jnp.minimum(lax.div(t, _bcast(ws)), FW - 1))
            gact.append((x2 > g * L) & (x1 < (g + 1) * L))
        params.append((x1, x2, y1, rank, h, hs, ixs, gact))

    def chunk_body(chunk, carry):
        c0 = chunk * CHUNK_W
        pltpu.sync_copy(front_h.at[pl.ds(chunk * FCHUNK_W, FCHUNK_W)], fbuf)
        for img in range(2):
            i = 2 * wid + img
            x1, x2, y1, rank, h, hs, ixs, gact = params[img]
            pltpu.sync_copy(back_h.at[i, pl.ds(c0, CHUNK_W)], buf)

            def row_body(k, rc, x1=x1, x2=x2, y1=y1, hs=hs, ixs=ixs, gact=gact):
                iy = jnp.minimum(lax.div(k * FH, hs), FH - 1)
                fb_row = iy * FW
                ob_row = (y1 + k) * W
                for g in range(W // L):
                    @pl.when(gact[g])
                    def _paste(g=g, fb_row=fb_row, ob_row=ob_row):
                        xpos = lanes + g * L
                        msk = (xpos >= x1) & (xpos < x2)
                        for cc in range(CB):
                            gidx = ixs[g] + (fb_row + cc * (FH * FW))
                            patch = plsc.load_gather(fbuf, [gidx])
                            sidx = xpos + (ob_row + cc * (H * W))
                            plsc.store_scatter(buf, [sidx], patch, mask=msk)
                return rc

            lax.fori_loop(0, h, row_body, jnp.int32(0))
            pltpu.sync_copy(buf, out_h.at[rank, pl.ds(c0, CHUNK_W)])
        return carry

    lax.fori_loop(0, NCHUNK, chunk_body, jnp.int32(0))


def kernel(front, back, rois):
    rois32 = rois.astype(jnp.int32).reshape(N * 4)
    front_f = front.reshape(C * FH * FW)
    back_f = back.reshape(N, IMG_W)
    mesh = plsc.VectorSubcoreMesh(core_axis_name="c", subcore_axis_name="s")
    out = pl.kernel(
        _body,
        out_type=jax.ShapeDtypeStruct((N, IMG_W), jnp.float32),
        mesh=mesh,
        compiler_params=pltpu.CompilerParams(needs_layout_passes=False),
        scratch_types=[
            pltpu.VMEM((N * 4,), jnp.int32),
            pltpu.VMEM((FCHUNK_W,), jnp.float32),
            pltpu.VMEM((CHUNK_W,), jnp.float32),
        ],
    )(front_f, back_f, rois32)
    return out.reshape(N, C, H, W)
